# baseline (speedup 1.0000x reference)
import functools

import jax
import jax.numpy as jnp
from jax import lax
from jax.experimental import pallas as pl
from jax.experimental.pallas import tpu as pltpu
from jax.experimental.pallas import tpu_sc as plsc

H = 128
W = 128
D = 128
B = 8
HW = H * W
DZ = 128.0 / D

NC = 2
NS = 16
NW = NC * NS

NSEG_PER_GROUP = 16
NGROUPS = HW // NSEG_PER_GROUP
GROUPS_PER_SUB = NGROUPS // NW
CHUNK_BLK = 4
NCHUNK = 5
SPAN_BLK = CHUNK_BLK * NCHUNK
PTR_PAD = HW + 1 + 15


def _make_sc_kernel(n_points: int):
    mesh = plsc.VectorSubcoreMesh(
        core_axis_name="c", subcore_axis_name="s", num_cores=NC,
        num_subcores=NS)

    @functools.partial(
        pl.kernel,
        mesh=mesh,
        compiler_params=pltpu.CompilerParams(
            needs_layout_passes=False, use_tc_tiling_on_sc=False),
        out_type=jax.ShapeDtypeStruct((B, HW), jnp.float32),
        scratch_types=[
            pltpu.VMEM((SPAN_BLK, 4, 128), jnp.float32),
            pltpu.VMEM((SPAN_BLK, 4, 128), jnp.float32),
            pltpu.VMEM((PTR_PAD,), jnp.int32),
            pltpu.VMEM((B * 4 * 16,), jnp.float32),
            pltpu.VMEM((B * NSEG_PER_GROUP * GROUPS_PER_SUB,), jnp.float32),
            pltpu.VMEM((48,), jnp.int32),
            pltpu.SemaphoreType.DMA,
            pltpu.SemaphoreType.DMA,
            pltpu.SemaphoreType.DMA,
        ],
    )
    def sc_kernel(grid_hbm, ptr_hbm, v_hbm, cols_hbm, out_hbm,
                  buf0, buf1, ptr_v, v_v, out_v, cols_v, sem0, sem1, sem_out):
        wid = lax.axis_index("s") * NC + lax.axis_index("c")
        pltpu.sync_copy(ptr_hbm, ptr_v)
        pltpu.sync_copy(v_hbm, v_v)
        pltpu.sync_copy(cols_hbm, cols_v)
        col0 = cols_v[pl.ds(0, 16)]
        col1 = cols_v[pl.ds(16, 16)]
        col2 = cols_v[pl.ds(32, 16)]

        iota = lax.iota(jnp.int32, 16)
        bufs = (buf0, buf1)
        sems = (sem0, sem1)
        nlim = n_points // 128 - SPAN_BLK

        def group_meta(t):
            g = NW * t + lax.rem(wid + 13 * t, NW)
            p_lo = ptr_v[pl.ds(16 * g, 16)]
            p_hi = plsc.load_gather(ptr_v, [16 * g + 1 + iota])
            s_lo = jnp.min(p_lo)
            s_hi = jnp.max(p_hi)
            start_blk = jnp.minimum(s_lo // 128, nlim)
            end_blk = (s_hi + 127) // 128
            nch = (end_blk - start_blk + CHUNK_BLK - 1) // CHUNK_BLK
            return g, p_lo, p_hi, start_blk, nch

        def chunk_copies(t, meta):
            _, _, _, start_blk, _ = meta
            buf = bufs[t % 2]
            return [pltpu.make_async_copy(
                grid_hbm.at[pl.ds(start_blk + CHUNK_BLK * i, CHUNK_BLK)],
                buf.at[pl.ds(CHUNK_BLK * i, CHUNK_BLK)], sems[t % 2])
                for i in range(NCHUNK)]

        def issue(t, meta):
            nch = meta[4]
            cps = chunk_copies(t, meta)
            for i, cp in enumerate(cps):
                if i < 3:
                    cp.start()
                else:
                    pl.when(i < nch)(cp.start)
            return cps

        def wait_chunks(meta, cps):
            nch = meta[4]
            for i, cp in enumerate(cps):
                if i < 3:
                    cp.wait()
                else:
                    pl.when(i < nch)(cp.wait)

        out_copies = []
        meta = group_meta(0)
        copy = issue(0, meta)
        for t in range(GROUPS_PER_SUB):
            nxt_meta = nxt_copy = None
            if t + 1 < GROUPS_PER_SUB:
                nxt_meta = group_meta(t + 1)
                nxt_copy = issue(t + 1, nxt_meta)
            g, p_lo, p_hi, start_blk, _ = meta
            buf = bufs[t % 2]
            wait_chunks(meta, copy)

            c = p_hi - p_lo
            cf = c.astype(jnp.float32)
            max_c = jnp.max(c)
            rel0 = p_lo - start_blk * 128
            zf = jnp.zeros((16,), jnp.float32)

            def body(k, carry):
                rel, ax, ay, az = carry
                m = c > k
                im = jnp.where(m, rel, 0)
                blk = lax.shift_right_logical(im, 7)
                off = lax.bitwise_and(im, 127)
                gx = plsc.load_gather(buf, [blk, col0, off])
                gy = plsc.load_gather(buf, [blk, col1, off])
                gz = plsc.load_gather(buf, [blk, col2, off])
                ax = ax + jnp.where(m, gx, zf)
                ay = ay + jnp.where(m, gy, zf)
                az = az + jnp.where(m, gz, zf)
                return (rel + 1, ax, ay, az)

            _, sx, sy, sz = lax.fori_loop(
                0, max_c, body, (rel0, zf, zf, zf))

            for b in range(B):
                vb0 = v_v[pl.ds((b * 4 + 0) * 16, 16)]
                vb1 = v_v[pl.ds((b * 4 + 1) * 16, 16)]
                vb2 = v_v[pl.ds((b * 4 + 2) * 16, 16)]
                vb3 = v_v[pl.ds((b * 4 + 3) * 16, 16)]
                ob = (vb0 * sx + vb1 * sy) + (vb2 * sz + vb3 * cf)
                slot = b * (NSEG_PER_GROUP * GROUPS_PER_SUB) + 16 * t
                out_v[pl.ds(slot, 16)] = ob
                out_copies.append(pltpu.async_copy(
                    out_v.at[pl.ds(slot, 16)],
                    out_hbm.at[b, pl.ds(16 * g, 16)], sem_out))

            meta, copy = nxt_meta, nxt_copy

        for oc in out_copies:
            oc.wait()

    return sc_kernel


def kernel(transform_matrix, grid, Wf, bf, segment_ids):
    n = grid.shape[0]
    seg = segment_ids.astype(jnp.int32)
    counts = jnp.zeros((HW,), jnp.int32).at[seg].add(1)
    ptr = jnp.concatenate([
        jnp.zeros((1,), jnp.int32),
        jnp.cumsum(counts).astype(jnp.int32),
        jnp.full((PTR_PAD - HW - 1,), n, jnp.int32)])
    cols = jnp.repeat(jnp.arange(3, dtype=jnp.int32), 16)
    v = jnp.einsum('bij,i->bj', transform_matrix[:, :3, :], Wf[:, 0])
    v = (v.at[:, 3].add(bf[0])) * jnp.float32(DZ)
    vpad = jnp.broadcast_to(v[:, :, None], (B, 4, 16)).reshape(-1)
    g3 = grid.reshape(n // 128, 128, 4).transpose(0, 2, 1)
    out = _make_sc_kernel(n)(g3, ptr, vpad, cols)
    return out.reshape(B, H, W)


# speedup vs baseline: 1.0329x; 1.0329x over previous
import functools

import jax
import jax.numpy as jnp
from jax import lax
from jax.experimental import pallas as pl
from jax.experimental.pallas import tpu as pltpu
from jax.experimental.pallas import tpu_sc as plsc

H = 128
W = 128
D = 128
B = 8
HW = H * W
DZ = 128.0 / D

NC = 2
NS = 16
NW = NC * NS

NSEG_PER_GROUP = 16
NGROUPS = HW // NSEG_PER_GROUP
GROUPS_PER_SUB = NGROUPS // NW
SPAN_BLK = 17
PTR_PAD = HW + 1 + 15


def _make_sc_kernel(n_points: int):
    mesh = plsc.VectorSubcoreMesh(
        core_axis_name="c", subcore_axis_name="s", num_cores=NC,
        num_subcores=NS)

    @functools.partial(
        pl.kernel,
        mesh=mesh,
        compiler_params=pltpu.CompilerParams(
            needs_layout_passes=False, use_tc_tiling_on_sc=False),
        out_type=jax.ShapeDtypeStruct((B, HW), jnp.float32),
        scratch_types=[
            pltpu.VMEM((SPAN_BLK, 4, 128), jnp.float32),
            pltpu.VMEM((SPAN_BLK, 4, 128), jnp.float32),
            pltpu.VMEM((SPAN_BLK, 4, 128), jnp.float32),
            pltpu.VMEM((PTR_PAD,), jnp.int32),
            pltpu.VMEM((B * 4 * 16,), jnp.float32),
            pltpu.VMEM((B * NSEG_PER_GROUP * GROUPS_PER_SUB,), jnp.float32),
            pltpu.VMEM((48,), jnp.int32),
            pltpu.SemaphoreType.DMA,
            pltpu.SemaphoreType.DMA,
            pltpu.SemaphoreType.DMA,
            pltpu.SemaphoreType.DMA,
        ],
    )
    def sc_kernel(grid_hbm, ptr_hbm, v_hbm, cols_hbm, out_hbm,
                  buf0, buf1, buf2, ptr_v, v_v, out_v, cols_v,
                  sem0, sem1, sem2, sem_out):
        wid = lax.axis_index("s") * NC + lax.axis_index("c")
        pltpu.sync_copy(ptr_hbm, ptr_v)
        pltpu.sync_copy(v_hbm, v_v)
        pltpu.sync_copy(cols_hbm, cols_v)
        col0 = cols_v[pl.ds(0, 16)]
        col1 = cols_v[pl.ds(16, 16)]
        col2 = cols_v[pl.ds(32, 16)]

        iota = lax.iota(jnp.int32, 16)
        bufs = (buf0, buf1, buf2)
        sems = (sem0, sem1, sem2)
        nlim = n_points // 128 - SPAN_BLK

        def group_meta(t):
            g = NW * t + lax.rem(wid + 13 * t, NW)
            p_lo = ptr_v[pl.ds(16 * g, 16)]
            p_hi = plsc.load_gather(ptr_v, [16 * g + 1 + iota])
            s_lo = jnp.min(p_lo)
            start_blk = jnp.minimum(s_lo // 128, nlim)
            return g, p_lo, p_hi, start_blk

        def issue(t, meta):
            _, _, _, start_blk = meta
            buf = bufs[t % 3]
            return pltpu.async_copy(
                grid_hbm.at[pl.ds(start_blk, SPAN_BLK)], buf, sems[t % 3])

        out_copies = []
        window = [(group_meta(0), None), (group_meta(1), None)]
        window[0] = (window[0][0], issue(0, window[0][0]))
        window[1] = (window[1][0], issue(1, window[1][0]))
        for t in range(GROUPS_PER_SUB):
            if t + 2 < GROUPS_PER_SUB:
                m2 = group_meta(t + 2)
                window.append((m2, issue(t + 2, m2)))
            meta, copy = window.pop(0)
            g, p_lo, p_hi, start_blk = meta
            buf = bufs[t % 3]
            copy.wait()

            c = p_hi - p_lo
            cf = c.astype(jnp.float32)
            max_c = jnp.max(c)
            rel0 = p_lo - start_blk * 128
            zf = jnp.zeros((16,), jnp.float32)

            def body(k, carry):
                rel, ax, ay, az = carry
                m = c > k
                im = jnp.where(m, rel, 0)
                blk = lax.shift_right_logical(im, 7)
                off = lax.bitwise_and(im, 127)
                gx = plsc.load_gather(buf, [blk, col0, off])
                gy = plsc.load_gather(buf, [blk, col1, off])
                gz = plsc.load_gather(buf, [blk, col2, off])
                ax = ax + jnp.where(m, gx, zf)
                ay = ay + jnp.where(m, gy, zf)
                az = az + jnp.where(m, gz, zf)
                return (rel + 1, ax, ay, az)

            _, sx, sy, sz = lax.fori_loop(
                0, max_c, body, (rel0, zf, zf, zf))

            for b in range(B):
                vb0 = v_v[pl.ds((b * 4 + 0) * 16, 16)]
                vb1 = v_v[pl.ds((b * 4 + 1) * 16, 16)]
                vb2 = v_v[pl.ds((b * 4 + 2) * 16, 16)]
                vb3 = v_v[pl.ds((b * 4 + 3) * 16, 16)]
                ob = (vb0 * sx + vb1 * sy) + (vb2 * sz + vb3 * cf)
                slot = b * (NSEG_PER_GROUP * GROUPS_PER_SUB) + 16 * t
                out_v[pl.ds(slot, 16)] = ob
                out_copies.append(pltpu.async_copy(
                    out_v.at[pl.ds(slot, 16)],
                    out_hbm.at[b, pl.ds(16 * g, 16)], sem_out))

        for oc in out_copies:
            oc.wait()

    return sc_kernel


def kernel(transform_matrix, grid, Wf, bf, segment_ids):
    n = grid.shape[0]
    seg = segment_ids.astype(jnp.int32)
    counts = jnp.zeros((HW,), jnp.int32).at[seg].add(1)
    ptr = jnp.concatenate([
        jnp.zeros((1,), jnp.int32),
        jnp.cumsum(counts).astype(jnp.int32),
        jnp.full((PTR_PAD - HW - 1,), n, jnp.int32)])
    cols = jnp.repeat(jnp.arange(3, dtype=jnp.int32), 16)
    v = jnp.einsum('bij,i->bj', transform_matrix[:, :3, :], Wf[:, 0])
    v = (v.at[:, 3].add(bf[0])) * jnp.float32(DZ)
    vpad = jnp.broadcast_to(v[:, :, None], (B, 4, 16)).reshape(-1)
    g3 = grid.reshape(n // 128, 128, 4).transpose(0, 2, 1)
    out = _make_sc_kernel(n)(g3, ptr, vpad, cols)
    return out.reshape(B, H, W)
